# Initial kernel scaffold; baseline (speedup 1.0000x reference)
import jax
import jax.numpy as jnp
from jax.experimental import pallas as pl


def kernel(image, bboxes, labels, points, regress_ranges):
    raise NotImplementedError("write your pallas kernel here")


# speedup vs baseline: 8.2187x; 8.2187x over previous
import jax
import jax.numpy as jnp
from jax.experimental import pallas as pl

_INF = 100000000.0
_PB = 512
_GPAD = 1024


def _fcos_block(data_ref, pts_ref, out_ref):
    xs = pts_ref[:, 0:1]
    ys = pts_ref[:, 1:2]
    ls = pts_ref[:, 2:3]
    us = pts_ref[:, 3:4]

    bx1 = data_ref[0:1, :]
    by1 = data_ref[1:2, :]
    bx2 = data_ref[2:3, :]
    by2 = data_ref[3:4, :]
    lab = data_ref[4:5, :]

    l = xs - bx1
    t = ys - by1
    r = bx2 - xs
    b = by2 - ys

    areas = (l + r) * (t + b)
    mind = jnp.minimum(jnp.minimum(l, t), jnp.minimum(r, b))
    maxd = jnp.maximum(jnp.maximum(l, t), jnp.maximum(r, b))
    ok = (mind > 0.0) & (ls <= maxd) & (maxd <= us)
    areas = jnp.where(ok, areas, _INF)

    mv = jnp.min(areas, axis=1, keepdims=True)
    iota = jax.lax.broadcasted_iota(jnp.int32, (_PB, _GPAD), 1)
    idx = jnp.min(jnp.where(areas == mv, iota, _GPAD),
                  axis=1, keepdims=True)
    onehot = iota == idx

    zero = jnp.zeros((), jnp.float32)
    l_s = jnp.sum(jnp.where(onehot, l, zero), axis=1, keepdims=True)
    t_s = jnp.sum(jnp.where(onehot, t, zero), axis=1, keepdims=True)
    r_s = jnp.sum(jnp.where(onehot, r, zero), axis=1, keepdims=True)
    b_s = jnp.sum(jnp.where(onehot, b, zero), axis=1, keepdims=True)
    lab_s = jnp.sum(jnp.where(onehot, lab, zero), axis=1, keepdims=True)

    cls = jnp.where(mv == _INF, zero, lab_s)
    cnt = jnp.sqrt((jnp.minimum(l_s, t_s) / jnp.maximum(l_s, t_s)) *
                   (jnp.minimum(r_s, b_s) / jnp.maximum(r_s, b_s)))

    out_ref[:, 0:1] = l_s
    out_ref[:, 1:2] = t_s
    out_ref[:, 2:3] = r_s
    out_ref[:, 3:4] = b_s
    out_ref[:, 4:5] = cls
    out_ref[:, 5:6] = cnt
    out_ref[:, 6:8] = jnp.zeros((_PB, 2), jnp.float32)


def kernel(image, bboxes, labels, points, regress_ranges):
    P = points.shape[0]
    G = bboxes.shape[0]
    p_pad = ((P + _PB - 1) // _PB) * _PB

    data = jnp.concatenate(
        [bboxes.T, labels.astype(jnp.float32)[None, :],
         jnp.zeros((3, G), jnp.float32)], axis=0)
    data = jnp.pad(data, ((0, 0), (0, _GPAD - G)))

    pts = jnp.concatenate([points, regress_ranges], axis=1)
    pts = jnp.pad(pts, ((0, p_pad - P), (0, 0)))

    out = pl.pallas_call(
        _fcos_block,
        grid=(p_pad // _PB,),
        in_specs=[
            pl.BlockSpec((8, _GPAD), lambda i: (0, 0)),
            pl.BlockSpec((_PB, 4), lambda i: (i, 0)),
        ],
        out_specs=pl.BlockSpec((_PB, 8), lambda i: (i, 0)),
        out_shape=jax.ShapeDtypeStruct((p_pad, 8), jnp.float32),
    )(data, pts)

    reg_targets = out[:P, 0:4]
    cls_targets = out[:P, 4].astype(jnp.int32)
    cnt_targets = out[:P, 5:6]
    return (image, reg_targets, cls_targets, cnt_targets)


# speedup vs baseline: 8.9963x; 1.0946x over previous
import jax
import jax.numpy as jnp
from jax.experimental import pallas as pl

_INF = 100000000.0
_PB = 512
_GPAD = 1024


def _fcos_block(data_ref, tab_ref, pts_ref, out_ref):
    xs = pts_ref[:, 0:1]
    ys = pts_ref[:, 1:2]
    ls = pts_ref[:, 2:3]
    us = pts_ref[:, 3:4]

    bx1 = data_ref[0:1, :]
    by1 = data_ref[1:2, :]
    bx2 = data_ref[2:3, :]
    by2 = data_ref[3:4, :]

    l = xs - bx1
    t = ys - by1
    r = bx2 - xs
    b = by2 - ys

    areas = (l + r) * (t + b)
    mind = jnp.minimum(jnp.minimum(l, t), jnp.minimum(r, b))
    maxd = jnp.maximum(jnp.maximum(l, t), jnp.maximum(r, b))
    ok = (mind > 0.0) & (ls <= maxd) & (maxd <= us)
    areas = jnp.where(ok, areas, _INF)

    mv = jnp.min(areas, axis=1, keepdims=True)
    iota = jax.lax.broadcasted_iota(jnp.int32, (_PB, _GPAD), 1)
    idx = jnp.min(jnp.where(areas == mv, iota, _GPAD),
                  axis=1, keepdims=True)
    onehot = jnp.where(iota == idx, 1.0, 0.0)

    sel = jax.lax.dot_general(
        onehot, tab_ref[...],
        dimension_numbers=(((1,), (0,)), ((), ())),
        preferred_element_type=jnp.float32)

    l_s = xs - sel[:, 0:1]
    t_s = ys - sel[:, 1:2]
    r_s = sel[:, 2:3] - xs
    b_s = sel[:, 3:4] - ys
    lab_s = sel[:, 4:5]

    zero = jnp.zeros((), jnp.float32)
    cls = jnp.where(mv == _INF, zero, lab_s)
    cnt = jnp.sqrt((jnp.minimum(l_s, t_s) / jnp.maximum(l_s, t_s)) *
                   (jnp.minimum(r_s, b_s) / jnp.maximum(r_s, b_s)))

    out_ref[:, 0:1] = l_s
    out_ref[:, 1:2] = t_s
    out_ref[:, 2:3] = r_s
    out_ref[:, 3:4] = b_s
    out_ref[:, 4:5] = cls
    out_ref[:, 5:6] = cnt
    out_ref[:, 6:8] = jnp.zeros((_PB, 2), jnp.float32)


def kernel(image, bboxes, labels, points, regress_ranges):
    P = points.shape[0]
    G = bboxes.shape[0]
    p_pad = ((P + _PB - 1) // _PB) * _PB

    data = jnp.concatenate(
        [bboxes.T, jnp.zeros((4, G), jnp.float32)], axis=0)
    data = jnp.pad(data, ((0, 0), (0, _GPAD - G)))

    tab = jnp.concatenate(
        [bboxes, labels.astype(jnp.float32)[:, None],
         jnp.zeros((G, 3), jnp.float32)], axis=1)
    tab = jnp.pad(tab, ((0, _GPAD - G), (0, 0)))

    pts = jnp.concatenate([points, regress_ranges], axis=1)
    pts = jnp.pad(pts, ((0, p_pad - P), (0, 0)))

    out = pl.pallas_call(
        _fcos_block,
        grid=(p_pad // _PB,),
        in_specs=[
            pl.BlockSpec((8, _GPAD), lambda i: (0, 0)),
            pl.BlockSpec((_GPAD, 8), lambda i: (0, 0)),
            pl.BlockSpec((_PB, 4), lambda i: (i, 0)),
        ],
        out_specs=pl.BlockSpec((_PB, 8), lambda i: (i, 0)),
        out_shape=jax.ShapeDtypeStruct((p_pad, 8), jnp.float32),
    )(data, tab, pts)

    reg_targets = out[:P, 0:4]
    cls_targets = out[:P, 4].astype(jnp.int32)
    cnt_targets = out[:P, 5:6]
    return (image, reg_targets, cls_targets, cnt_targets)
